# Initial kernel scaffold; baseline (speedup 1.0000x reference)
import jax
import jax.numpy as jnp
from jax.experimental import pallas as pl


def kernel(X):
    raise NotImplementedError("write your pallas kernel here")


# speedup vs baseline: 812.9056x; 812.9056x over previous
import functools

import jax
import jax.numpy as jnp
from jax import lax
from jax.experimental import pallas as pl
from jax.experimental.pallas import tpu as pltpu
from jax.experimental.pallas import tpu_sc as plsc

N = 10000
N_PAD = 10240
ROWS = N_PAD // 128
IB = 128
NUM_TILES = 16
PER_TILE = N_PAD // NUM_TILES
ROWS_PER_TILE = PER_TILE // 128


def _rank_body(xcol_ref, xrow_ref, out_ref):
    xi = jnp.broadcast_to(xcol_ref[...], (IB, 128))

    def body(k, acc):
        row = xrow_ref[pl.ds(k, 1), :]
        return acc + (row < xi).astype(jnp.float32)

    acc = lax.fori_loop(0, ROWS, body, jnp.zeros((IB, 128), jnp.float32))
    out_ref[...] = jnp.sum(acc, axis=1, keepdims=True).astype(jnp.int32)


def _ranks(xpad):
    xcol = xpad.reshape(N_PAD, 1)
    xrow = xpad.reshape(ROWS, 128)
    grid = N_PAD // IB
    ranks = pl.pallas_call(
        _rank_body,
        grid=(grid,),
        in_specs=[
            pl.BlockSpec((IB, 1), lambda i: (i, 0)),
            pl.BlockSpec((ROWS, 128), lambda i: (0, 0)),
        ],
        out_specs=pl.BlockSpec((IB, 1), lambda i: (i, 0)),
        out_shape=jax.ShapeDtypeStruct((N_PAD, 1), jnp.int32),
    )(xcol, xrow)
    return ranks.reshape(NUM_TILES, ROWS_PER_TILE, 128)


def _sc_body(x_hbm, r2d_hbm, j2d_hbm, z_hbm, out_hbm,
             ridx_v, vals_v, y3_v, gidx_v, out_v, y3_sh):
    t = lax.axis_index("s")
    base = t * PER_TILE

    pltpu.sync_copy(z_hbm.at[pl.ds(base, PER_TILE)],
                    y3_sh.at[pl.ds(base, PER_TILE)])
    plsc.subcore_barrier()

    pltpu.sync_copy(r2d_hbm.at[t], ridx_v)
    pltpu.sync_copy(j2d_hbm.at[t], vals_v)
    for r in range(ROWS_PER_TILE):
        pltpu.sync_copy(vals_v.at[r], y3_sh.at[ridx_v.at[r]], add=True)
    plsc.subcore_barrier()

    pltpu.sync_copy(y3_sh.at[pl.ds(base, PER_TILE)], y3_v)
    nvec = jnp.full((16,), N, jnp.int32)
    for k in range(PER_TILE // 16):
        v = y3_v[pl.ds(k * 16, 16)]
        gidx_v[pl.ds(k * 16, 16)] = lax.rem(v, nvec)
    for r in range(ROWS_PER_TILE):
        pltpu.sync_copy(x_hbm.at[gidx_v.at[pl.ds(r * 128, 128)]],
                        out_v.at[pl.ds(r * 128, 128)])
    pltpu.sync_copy(out_v, out_hbm.at[pl.ds(base, PER_TILE)])


def _sc_sort(xpad, ranks2d, jidx2d, zeros1d):
    mesh = plsc.VectorSubcoreMesh(core_axis_name="c", subcore_axis_name="s",
                                  num_cores=1)
    k = functools.partial(
        pl.kernel,
        out_type=jax.ShapeDtypeStruct((N_PAD,), jnp.float32),
        mesh=mesh,
        scratch_types=[
            pltpu.VMEM((ROWS_PER_TILE, 128), jnp.int32),
            pltpu.VMEM((ROWS_PER_TILE, 128), jnp.int32),
            pltpu.VMEM((PER_TILE,), jnp.int32),
            pltpu.VMEM((PER_TILE,), jnp.int32),
            pltpu.VMEM((PER_TILE,), jnp.float32),
            pltpu.VMEM_SHARED((N_PAD,), jnp.int32),
        ],
    )(_sc_body)
    return k(xpad, ranks2d, jidx2d, zeros1d)


def kernel(X):
    x = X[0]
    xpad = jnp.concatenate([x, jnp.full((N_PAD - N,), jnp.inf, jnp.float32)])
    ranks2d = _ranks(xpad)
    jidx2d = jnp.arange(N_PAD, dtype=jnp.int32).reshape(
        NUM_TILES, ROWS_PER_TILE, 128)
    zeros1d = jnp.zeros((N_PAD,), jnp.int32)
    out = _sc_sort(xpad, ranks2d, jidx2d, zeros1d)
    return out[:N]


# speedup vs baseline: 921.1101x; 1.1331x over previous
import functools

import jax
import jax.numpy as jnp
from jax import lax
from jax.experimental import pallas as pl
from jax.experimental.pallas import tpu as pltpu
from jax.experimental.pallas import tpu_sc as plsc

N = 10000
N_PAD = 10240
ROWS = N_PAD // 128
IB = 128
NUM_TILES = 16
PER_TILE = N_PAD // NUM_TILES
ROWS_PER_TILE = PER_TILE // 128


def _rank_body(xcol_ref, xrow_ref, out_ref):
    xi = jnp.broadcast_to(xcol_ref[...], (IB, 128))

    def body(k, acc):
        row = xrow_ref[pl.ds(k, 1), :]
        return acc + (row < xi).astype(jnp.float32)

    acc = lax.fori_loop(0, ROWS, body, jnp.zeros((IB, 128), jnp.float32),
                        unroll=8)
    out_ref[...] = jnp.sum(acc, axis=1, keepdims=True).astype(jnp.int32)


def _ranks(xpad):
    xcol = xpad.reshape(N_PAD, 1)
    xrow = xpad.reshape(ROWS, 128)
    grid = N_PAD // IB
    ranks = pl.pallas_call(
        _rank_body,
        grid=(grid,),
        in_specs=[
            pl.BlockSpec((IB, 1), lambda i: (i, 0)),
            pl.BlockSpec((ROWS, 128), lambda i: (0, 0)),
        ],
        out_specs=pl.BlockSpec((IB, 1), lambda i: (i, 0)),
        out_shape=jax.ShapeDtypeStruct((N_PAD, 1), jnp.int32),
    )(xcol, xrow)
    return ranks.reshape(NUM_TILES, ROWS_PER_TILE, 128)


def _sc_body(x_hbm, r2d_hbm, j2d_hbm, z_hbm, out_hbm,
             ridx_v, vals_v, y3_v, gidx_v, out_v, y3_sh):
    t = lax.axis_index("s")
    base = t * PER_TILE

    pltpu.sync_copy(z_hbm.at[pl.ds(base, PER_TILE)],
                    y3_sh.at[pl.ds(base, PER_TILE)])
    plsc.subcore_barrier()

    pltpu.sync_copy(r2d_hbm.at[t], ridx_v)
    pltpu.sync_copy(j2d_hbm.at[t], vals_v)
    for r in range(ROWS_PER_TILE):
        pltpu.sync_copy(vals_v.at[r], y3_sh.at[ridx_v.at[r]], add=True)
    plsc.subcore_barrier()

    pltpu.sync_copy(y3_sh.at[pl.ds(base, PER_TILE)], y3_v)
    nvec = jnp.full((16,), N, jnp.int32)
    for k in range(PER_TILE // 16):
        v = y3_v[pl.ds(k * 16, 16)]
        gidx_v[pl.ds(k * 16, 16)] = lax.rem(v, nvec)
    for r in range(ROWS_PER_TILE):
        pltpu.sync_copy(x_hbm.at[gidx_v.at[pl.ds(r * 128, 128)]],
                        out_v.at[pl.ds(r * 128, 128)])
    pltpu.sync_copy(out_v, out_hbm.at[pl.ds(base, PER_TILE)])


def _sc_sort(xpad, ranks2d, jidx2d, zeros1d):
    mesh = plsc.VectorSubcoreMesh(core_axis_name="c", subcore_axis_name="s",
                                  num_cores=1)
    k = functools.partial(
        pl.kernel,
        out_type=jax.ShapeDtypeStruct((N_PAD,), jnp.float32),
        mesh=mesh,
        scratch_types=[
            pltpu.VMEM((ROWS_PER_TILE, 128), jnp.int32),
            pltpu.VMEM((ROWS_PER_TILE, 128), jnp.int32),
            pltpu.VMEM((PER_TILE,), jnp.int32),
            pltpu.VMEM((PER_TILE,), jnp.int32),
            pltpu.VMEM((PER_TILE,), jnp.float32),
            pltpu.VMEM_SHARED((N_PAD,), jnp.int32),
        ],
    )(_sc_body)
    return k(xpad, ranks2d, jidx2d, zeros1d)


def kernel(X):
    x = X[0]
    xpad = jnp.concatenate([x, jnp.full((N_PAD - N,), jnp.inf, jnp.float32)])
    ranks2d = _ranks(xpad)
    jidx2d = jnp.arange(N_PAD, dtype=jnp.int32).reshape(
        NUM_TILES, ROWS_PER_TILE, 128)
    zeros1d = jnp.zeros((N_PAD,), jnp.int32)
    out = _sc_sort(xpad, ranks2d, jidx2d, zeros1d)
    return out[:N]


# speedup vs baseline: 946.6666x; 1.0277x over previous
import functools

import jax
import jax.numpy as jnp
from jax import lax
from jax.experimental import pallas as pl
from jax.experimental.pallas import tpu as pltpu
from jax.experimental.pallas import tpu_sc as plsc

N = 10000
N_PAD = 10240
ROWS = N_PAD // 128
IB = 256
NUM_TILES = 16
PER_TILE = N_PAD // NUM_TILES
ROWS_PER_TILE = PER_TILE // 128


def _rank_body(xcol_ref, xrow_ref, out_ref):
    xi = jnp.broadcast_to(xcol_ref[...], (IB, 128))

    def body(k, acc):
        row = xrow_ref[pl.ds(k, 1), :]
        return acc + (row < xi).astype(jnp.float32)

    acc = lax.fori_loop(0, ROWS, body, jnp.zeros((IB, 128), jnp.float32),
                        unroll=4)
    out_ref[...] = jnp.sum(acc, axis=1, keepdims=True).astype(jnp.int32)


def _ranks(xpad):
    xcol = xpad.reshape(N_PAD, 1)
    xrow = xpad.reshape(ROWS, 128)
    grid = N_PAD // IB
    ranks = pl.pallas_call(
        _rank_body,
        grid=(grid,),
        in_specs=[
            pl.BlockSpec((IB, 1), lambda i: (i, 0)),
            pl.BlockSpec((ROWS, 128), lambda i: (0, 0)),
        ],
        out_specs=pl.BlockSpec((IB, 1), lambda i: (i, 0)),
        out_shape=jax.ShapeDtypeStruct((N_PAD, 1), jnp.int32),
    )(xcol, xrow)
    return ranks.reshape(NUM_TILES, ROWS_PER_TILE, 128)


def _sc_body(x_hbm, r2d_hbm, j2d_hbm, z_hbm, out_hbm,
             ridx_v, vals_v, y3_v, gidx_v, out_v, y3_sh):
    t = lax.axis_index("s")
    base = t * PER_TILE

    pltpu.sync_copy(z_hbm.at[pl.ds(base, PER_TILE)],
                    y3_sh.at[pl.ds(base, PER_TILE)])
    plsc.subcore_barrier()

    pltpu.sync_copy(r2d_hbm.at[t], ridx_v)
    pltpu.sync_copy(j2d_hbm.at[t], vals_v)
    for r in range(ROWS_PER_TILE):
        pltpu.sync_copy(vals_v.at[r], y3_sh.at[ridx_v.at[r]], add=True)
    plsc.subcore_barrier()

    pltpu.sync_copy(y3_sh.at[pl.ds(base, PER_TILE)], y3_v)
    nvec = jnp.full((16,), N, jnp.int32)
    for k in range(PER_TILE // 16):
        v = y3_v[pl.ds(k * 16, 16)]
        gidx_v[pl.ds(k * 16, 16)] = lax.rem(v, nvec)
    for r in range(ROWS_PER_TILE):
        pltpu.sync_copy(x_hbm.at[gidx_v.at[pl.ds(r * 128, 128)]],
                        out_v.at[pl.ds(r * 128, 128)])
    pltpu.sync_copy(out_v, out_hbm.at[pl.ds(base, PER_TILE)])


def _sc_sort(xpad, ranks2d, jidx2d, zeros1d):
    mesh = plsc.VectorSubcoreMesh(core_axis_name="c", subcore_axis_name="s",
                                  num_cores=1)
    k = functools.partial(
        pl.kernel,
        out_type=jax.ShapeDtypeStruct((N_PAD,), jnp.float32),
        mesh=mesh,
        scratch_types=[
            pltpu.VMEM((ROWS_PER_TILE, 128), jnp.int32),
            pltpu.VMEM((ROWS_PER_TILE, 128), jnp.int32),
            pltpu.VMEM((PER_TILE,), jnp.int32),
            pltpu.VMEM((PER_TILE,), jnp.int32),
            pltpu.VMEM((PER_TILE,), jnp.float32),
            pltpu.VMEM_SHARED((N_PAD,), jnp.int32),
        ],
    )(_sc_body)
    return k(xpad, ranks2d, jidx2d, zeros1d)


def kernel(X):
    x = X[0]
    xpad = jnp.concatenate([x, jnp.full((N_PAD - N,), jnp.inf, jnp.float32)])
    ranks2d = _ranks(xpad)
    jidx2d = jnp.arange(N_PAD, dtype=jnp.int32).reshape(
        NUM_TILES, ROWS_PER_TILE, 128)
    zeros1d = jnp.zeros((N_PAD,), jnp.int32)
    out = _sc_sort(xpad, ranks2d, jidx2d, zeros1d)
    return out[:N]
